# Initial kernel scaffold; baseline (speedup 1.0000x reference)
import jax
import jax.numpy as jnp
from jax.experimental import pallas as pl


def kernel(data, size_table):
    raise NotImplementedError("write your pallas kernel here")


# speedup vs baseline: 2.4128x; 2.4128x over previous
import functools

import jax
import jax.numpy as jnp
from jax import lax
from jax.experimental import pallas as pl
from jax.experimental.pallas import tpu as pltpu
from jax.experimental.pallas import tpu_sc as plsc

EMBED = 64
MAXSIZE = 520
B = 4096
L = 50

NUM_CH = 6
TOTAL_ROWS = B * L * NUM_CH
NC, NS = 2, 16
NW = NC * NS
ROWS_PER_W = TOTAL_ROWS // NW
CHUNK = 768
N_CHUNKS = ROWS_PER_W // CHUNK


def _build_table(size_table):
    def body(st_ref, out_ref):
        out_ref[0:MAXSIZE, :] = st_ref[...]
        pos = lax.broadcasted_iota(jnp.int32, (MAXSIZE, EMBED), 0).astype(jnp.float32)
        col = lax.broadcasted_iota(jnp.int32, (MAXSIZE, EMBED), 1)
        j = (col % (EMBED // 2)).astype(jnp.float32)
        freq = jnp.exp(-jnp.log(10000.0) * (2.0 * j) / EMBED)
        ang = pos * freq
        out_ref[MAXSIZE:2 * MAXSIZE, :] = jnp.where(
            col < EMBED // 2, jnp.sin(ang), jnp.cos(ang))

    return pl.pallas_call(
        body,
        out_shape=jax.ShapeDtypeStruct((2 * MAXSIZE, EMBED), jnp.float32),
    )(size_table)


def _make_sc_gather():
    mesh = plsc.VectorSubcoreMesh(
        core_axis_name="c", subcore_axis_name="s",
        num_cores=NC, num_subcores=NS)

    @functools.partial(
        pl.kernel,
        out_type=jax.ShapeDtypeStruct((TOTAL_ROWS, EMBED), jnp.float32),
        mesh=mesh,
        scratch_types=[
            pltpu.VMEM((CHUNK,), jnp.int32),
            pltpu.VMEM((CHUNK, EMBED), jnp.float32),
            pltpu.SemaphoreType.DMA,
        ],
        compiler_params=pltpu.CompilerParams(use_tc_tiling_on_sc=False),
    )
    def sc_gather(table_hbm, data_hbm, out_hbm, idx_v, rows_v, sem):
        wid = lax.axis_index("s") * NC + lax.axis_index("c")
        base = wid * ROWS_PER_W

        lane = lax.iota(jnp.int32, 16)
        offs = {
            ph: jnp.where((lane + ph) % NUM_CH == 0, 0, MAXSIZE).astype(jnp.int32)
            for ph in (0, 2, 4)
        }

        def step(i, carry):
            row0 = base + i * CHUNK
            pltpu.sync_copy(data_hbm.at[pl.ds(row0, CHUNK)], idx_v)
            for j in range(CHUNK // 16):
                ph = (j * 16) % NUM_CH
                sl = pl.ds(j * 16, 16)
                idx_v[sl] = idx_v[sl] + offs[ph]
            pltpu.async_copy(table_hbm.at[idx_v], rows_v, sem).wait()
            pltpu.sync_copy(rows_v, out_hbm.at[pl.ds(row0, CHUNK)])
            return carry

        lax.fori_loop(0, N_CHUNKS, step, 0)

    return sc_gather


_sc_gather = _make_sc_gather()


def kernel(data, size_table):
    table = _build_table(size_table)
    data_flat = data.reshape(TOTAL_ROWS)
    out = _sc_gather(table, data_flat)
    return out.reshape(B, L, NUM_CH * EMBED)
